# baseline (speedup 1.0000x reference)
import functools
import jax
import jax.numpy as jnp
from jax import lax
from jax.experimental import pallas as pl
from jax.experimental.pallas import tpu as pltpu
from jax.experimental.pallas import tpu_sc as plsc

NC, NS, L = 2, 16, 16
NW = NC * NS
B = 16384
V = 100001
UD = 32
SD = 8
OD = UD + 2 * SD
CHUNK = 4096
N_CH = B // CHUNK
HALF = B // 2
STAGE = V + 7

_mesh = plsc.VectorSubcoreMesh(core_axis_name="c", subcore_axis_name="s")


@functools.partial(
    pl.kernel,
    out_type=jax.ShapeDtypeStruct((OD, B), jnp.float32),
    mesh=_mesh,
    compiler_params=pltpu.CompilerParams(use_tc_tiling_on_sc=False,
                                         needs_layout_passes=False),
    scratch_types=[
        pltpu.VMEM((STAGE,), jnp.float32),
        pltpu.VMEM((2 * CHUNK,), jnp.int32),
        pltpu.VMEM((HALF,), jnp.int32),
        pltpu.VMEM((2 * CHUNK,), jnp.float32),
        pltpu.VMEM((3, SD), jnp.float32),
        pltpu.VMEM((22, SD), jnp.float32),
        pltpu.SemaphoreType.DMA,
        pltpu.SemaphoreType.DMA,
        pltpu.SemaphoreType.DMA,
        pltpu.SemaphoreType.DMA,
        pltpu.SemaphoreType.DMA,
    ],
)
def _user_model_sc(uid_hbm, gid_hbm, oid_hbm, utabf_hbm, gtab_hbm, otab_hbm,
                   out_hbm, row_v, idx_v, sidx_v, obuf_v, gt_v, ot_v,
                   sem_row, sem_u0, sem_u1, sem_s, sem_o):
    wid = lax.axis_index("s") * NC + lax.axis_index("c")
    sems_u = (sem_u0, sem_u1)

    row_begin = wid * V
    start = pl.multiple_of((row_begin // 8) * 8, 8)
    misal = row_begin - start
    cp_row = pltpu.async_copy(utabf_hbm.at[pl.ds(start, STAGE)], row_v,
                              sem_row)

    cp_u = [None] * N_CH
    for k in range(2):
        cp_u[k] = pltpu.async_copy(uid_hbm.at[pl.ds(k * CHUNK, CHUNK)],
                                   idx_v.at[pl.ds(k * CHUNK, CHUNK)],
                                   sems_u[k])

    d = wid % 16
    col0 = (wid // 16) * HALF
    pltpu.sync_copy(gtab_hbm, gt_v)
    pltpu.sync_copy(otab_hbm, ot_v)

    def small(src_hbm, tab_v, dim):
        pltpu.async_copy(src_hbm.at[pl.ds(col0, HALF)], sidx_v, sem_s).wait()
        dvec = jnp.broadcast_to(dim, (L,))

        @plsc.parallel_loop(0, HALF, step=L, unroll=8)
        def body(j):
            ids = sidx_v[pl.ds(j, L)]
            obuf_v[pl.ds(j, L)] = plsc.load_gather(tab_v, [ids, dvec])

    @pl.when(d < SD)
    def _gender():
        small(gid_hbm, gt_v, d)

    @pl.when(d >= SD)
    def _occ():
        small(oid_hbm, ot_v, d - SD)

    cp_so = pltpu.async_copy(obuf_v, out_hbm.at[UD + d, pl.ds(col0, HALF)],
                             sem_o)

    cp_row.wait()
    cp_so.wait()
    cp_o = [None] * N_CH
    for k in range(N_CH):
        p = k % 2
        cp_u[k].wait()
        if k >= 2:
            cp_o[k - 2].wait()

        @plsc.parallel_loop(0, CHUNK, step=L, unroll=8)
        def mbody(j):
            ids = idx_v[pl.ds(p * CHUNK + j, L)] + misal
            obuf_v[pl.ds(p * CHUNK + j, L)] = plsc.load_gather(row_v, [ids])
        cp_o[k] = pltpu.async_copy(obuf_v.at[pl.ds(p * CHUNK, CHUNK)],
                                   out_hbm.at[wid, pl.ds(k * CHUNK, CHUNK)],
                                   sem_o)
        if k + 2 < N_CH:
            cp_u[k + 2] = pltpu.async_copy(
                uid_hbm.at[pl.ds((k + 2) * CHUNK, CHUNK)],
                idx_v.at[pl.ds(p * CHUNK, CHUNK)], sems_u[p])
    cp_o[N_CH - 2].wait()
    cp_o[N_CH - 1].wait()


def kernel(user_id, gender, occupation, user_table, gender_table,
           occupation_table):
    utab_flat = user_table.T.reshape(UD * V)
    out_t = _user_model_sc(user_id, gender, occupation, utab_flat,
                           gender_table, occupation_table)
    return out_t.T


# speedup vs baseline: 1.1030x; 1.1030x over previous
import functools
import jax
import jax.numpy as jnp
from jax import lax
from jax.experimental import pallas as pl
from jax.experimental.pallas import tpu as pltpu
from jax.experimental.pallas import tpu_sc as plsc

NC, NS, L = 2, 16, 16
NW = NC * NS
B = 16384
V = 100001
UD = 32
SD = 8
OD = UD + 2 * SD
CHUNK = 4096
N_CH = B // CHUNK
HALF = B // 2
STAGE = V + 7
JB = 128
TI = OD // 8
TJ = B // JB

_mesh = plsc.VectorSubcoreMesh(core_axis_name="c", subcore_axis_name="s")


@functools.partial(
    pl.kernel,
    out_type=jax.ShapeDtypeStruct((TI, TJ, 8, JB), jnp.float32),
    mesh=_mesh,
    compiler_params=pltpu.CompilerParams(use_tc_tiling_on_sc=False,
                                         needs_layout_passes=False),
    scratch_types=[
        pltpu.VMEM((STAGE,), jnp.float32),
        pltpu.VMEM((2 * CHUNK,), jnp.int32),
        pltpu.VMEM((HALF,), jnp.int32),
        pltpu.VMEM((64, JB), jnp.float32),
        pltpu.VMEM((3, SD), jnp.float32),
        pltpu.VMEM((22, SD), jnp.float32),
        pltpu.SemaphoreType.DMA,
        pltpu.SemaphoreType.DMA,
        pltpu.SemaphoreType.DMA,
        pltpu.SemaphoreType.DMA,
        pltpu.SemaphoreType.DMA,
    ],
)
def _user_model_sc(uid_hbm, gid_hbm, oid_hbm, utabf_hbm, gtab_hbm, otab_hbm,
                   out_hbm, row_v, idx_v, sidx_v, obuf_v, gt_v, ot_v,
                   sem_row, sem_u0, sem_u1, sem_s, sem_o):
    wid = lax.axis_index("s") * NC + lax.axis_index("c")
    sems_u = (sem_u0, sem_u1)

    row_begin = wid * V
    start = pl.multiple_of((row_begin // 8) * 8, 8)
    misal = row_begin - start
    cp_row = pltpu.async_copy(utabf_hbm.at[pl.ds(start, STAGE)], row_v,
                              sem_row)

    cp_u = [None] * N_CH
    for k in range(2):
        cp_u[k] = pltpu.async_copy(uid_hbm.at[pl.ds(k * CHUNK, CHUNK)],
                                   idx_v.at[pl.ds(k * CHUNK, CHUNK)],
                                   sems_u[k])

    d = wid % 16
    col0 = (wid // 16) * HALF
    j0 = (wid // 16) * (HALF // JB)
    pltpu.sync_copy(gtab_hbm, gt_v)
    pltpu.sync_copy(otab_hbm, ot_v)

    def small(src_hbm, tab_v, dim):
        pltpu.async_copy(src_hbm.at[pl.ds(col0, HALF)], sidx_v, sem_s).wait()
        dvec = jnp.broadcast_to(dim, (L,))

        @plsc.parallel_loop(0, HALF // L, unroll=8)
        def body(m):
            ids = sidx_v[pl.ds(m * L, L)]
            obuf_v[m // 8, pl.ds((m % 8) * L, L)] = plsc.load_gather(
                tab_v, [ids, dvec])

    @pl.when(d < SD)
    def _gender():
        small(gid_hbm, gt_v, d)

    @pl.when(d >= SD)
    def _occ():
        small(oid_hbm, ot_v, d - SD)

    r = UD + d
    cp_so = pltpu.async_copy(
        obuf_v, out_hbm.at[r // 8, pl.ds(j0, HALF // JB), r % 8], sem_o)

    cp_row.wait()
    cp_so.wait()
    cp_o = [None] * N_CH
    for k in range(N_CH):
        p = k % 2
        cp_u[k].wait()
        if k >= 2:
            cp_o[k - 2].wait()

        @plsc.parallel_loop(0, CHUNK // L, unroll=8)
        def mbody(m):
            ids = idx_v[pl.ds(p * CHUNK + m * L, L)] + misal
            obuf_v[p * 32 + m // 8, pl.ds((m % 8) * L, L)] = (
                plsc.load_gather(row_v, [ids]))

        cp_o[k] = pltpu.async_copy(
            obuf_v.at[pl.ds(p * 32, 32)],
            out_hbm.at[wid // 8, pl.ds(k * 32, 32), wid % 8], sem_o)
        if k + 2 < N_CH:
            cp_u[k + 2] = pltpu.async_copy(
                uid_hbm.at[pl.ds((k + 2) * CHUNK, CHUNK)],
                idx_v.at[pl.ds(p * CHUNK, CHUNK)], sems_u[p])
    cp_o[N_CH - 2].wait()
    cp_o[N_CH - 1].wait()


def kernel(user_id, gender, occupation, user_table, gender_table,
           occupation_table):
    utab_flat = user_table.T.reshape(UD * V)
    out4 = _user_model_sc(user_id, gender, occupation, utab_flat,
                          gender_table, occupation_table)
    return out4.transpose(1, 3, 0, 2).reshape(B, OD)


# speedup vs baseline: 1.1037x; 1.0006x over previous
import functools
import jax
import jax.numpy as jnp
from jax import lax
from jax.experimental import pallas as pl
from jax.experimental.pallas import tpu as pltpu
from jax.experimental.pallas import tpu_sc as plsc

NC, NS, L = 2, 16, 16
NW = NC * NS
B = 16384
V = 100001
UD = 32
SD = 8
OD = UD + 2 * SD
CHUNK = 4096
N_CH = B // CHUNK
HALF = B // 2
STAGE = V + 7
JB = 128
TI = OD // 8
TJ = B // JB

_mesh = plsc.VectorSubcoreMesh(core_axis_name="c", subcore_axis_name="s")


@functools.partial(
    pl.kernel,
    out_type=jax.ShapeDtypeStruct((TI, TJ, 8, JB), jnp.float32),
    mesh=_mesh,
    compiler_params=pltpu.CompilerParams(use_tc_tiling_on_sc=False,
                                         needs_layout_passes=False),
    scratch_types=[
        pltpu.VMEM((STAGE,), jnp.float32),
        pltpu.VMEM((2 * CHUNK,), jnp.int32),
        pltpu.VMEM((HALF,), jnp.int32),
        pltpu.VMEM((64, JB), jnp.float32),
        pltpu.VMEM((3, SD), jnp.float32),
        pltpu.VMEM((22, SD), jnp.float32),
        pltpu.SemaphoreType.DMA,
        pltpu.SemaphoreType.DMA,
        pltpu.SemaphoreType.DMA,
        pltpu.SemaphoreType.DMA,
        pltpu.SemaphoreType.DMA,
    ],
)
def _user_model_sc(uid_hbm, gid_hbm, oid_hbm, utabf_hbm, gtab_hbm, otab_hbm,
                   out_hbm, row_v, idx_v, sidx_v, obuf_v, gt_v, ot_v,
                   sem_row, sem_u0, sem_u1, sem_s, sem_o):
    wid = lax.axis_index("s") * NC + lax.axis_index("c")
    sems_u = (sem_u0, sem_u1)

    row_begin = wid * V
    start = pl.multiple_of((row_begin // 8) * 8, 8)
    misal = row_begin - start
    cp_row = pltpu.async_copy(utabf_hbm.at[pl.ds(start, STAGE)], row_v,
                              sem_row)

    cp_u = [None] * N_CH
    for k in range(2):
        cp_u[k] = pltpu.async_copy(uid_hbm.at[pl.ds(k * CHUNK, CHUNK)],
                                   idx_v.at[pl.ds(k * CHUNK, CHUNK)],
                                   sems_u[k])

    d = wid % 16
    col0 = (wid // 16) * HALF
    j0 = (wid // 16) * (HALF // JB)
    cp_gt = pltpu.async_copy(gtab_hbm, gt_v, sem_s)
    cp_ot = pltpu.async_copy(otab_hbm, ot_v, sem_s)

    def small(src_hbm, tab_v, dim):
        cp_si = pltpu.async_copy(src_hbm.at[pl.ds(col0, HALF)], sidx_v, sem_s)
        cp_gt.wait()
        cp_ot.wait()
        cp_si.wait()
        dvec = jnp.broadcast_to(dim, (L,))

        @plsc.parallel_loop(0, HALF // L, unroll=8)
        def body(m):
            ids = sidx_v[pl.ds(m * L, L)]
            obuf_v[m // 8, pl.ds((m % 8) * L, L)] = plsc.load_gather(
                tab_v, [ids, dvec])

    @pl.when(d < SD)
    def _gender():
        small(gid_hbm, gt_v, d)

    @pl.when(d >= SD)
    def _occ():
        small(oid_hbm, ot_v, d - SD)

    r = UD + d
    cp_so = [
        pltpu.async_copy(obuf_v.at[pl.ds(h * 32, 32)],
                         out_hbm.at[r // 8, pl.ds(j0 + h * 32, 32), r % 8],
                         sem_o)
        for h in range(2)
    ]

    cp_row.wait()
    cp_o = [None] * N_CH
    for k in range(N_CH):
        p = k % 2
        cp_u[k].wait()
        if k < 2:
            cp_so[k].wait()
        else:
            cp_o[k - 2].wait()

        @plsc.parallel_loop(0, CHUNK // L, unroll=8)
        def mbody(m):
            ids = idx_v[pl.ds(p * CHUNK + m * L, L)] + misal
            obuf_v[p * 32 + m // 8, pl.ds((m % 8) * L, L)] = (
                plsc.load_gather(row_v, [ids]))

        cp_o[k] = pltpu.async_copy(
            obuf_v.at[pl.ds(p * 32, 32)],
            out_hbm.at[wid // 8, pl.ds(k * 32, 32), wid % 8], sem_o)
        if k + 2 < N_CH:
            cp_u[k + 2] = pltpu.async_copy(
                uid_hbm.at[pl.ds((k + 2) * CHUNK, CHUNK)],
                idx_v.at[pl.ds(p * CHUNK, CHUNK)], sems_u[p])
    cp_o[N_CH - 2].wait()
    cp_o[N_CH - 1].wait()


def kernel(user_id, gender, occupation, user_table, gender_table,
           occupation_table):
    utab_flat = user_table.T.reshape(UD * V)
    out4 = _user_model_sc(user_id, gender, occupation, utab_flat,
                          gender_table, occupation_table)
    return out4.transpose(1, 3, 0, 2).reshape(B, OD)
